# baseline (speedup 1.0000x reference)
import functools

import jax
import jax.numpy as jnp
from jax import lax
from jax.experimental import pallas as pl
from jax.experimental.pallas import tpu as pltpu

_EPS = 1e-12


def _round_up(v, n):
    return (v + n - 1) // n * n


def _cosface_fused_kernel(lab_ref, x_ref, w_ref, o_ref, wn_ref,
                          *, s, m, tile_c):
    @pl.when(pl.program_id(1) == 0)
    def _():
        w = w_ref[...]
        sw = jnp.sum(w * w, axis=1, keepdims=True)
        inv_nw = lax.rsqrt(jnp.maximum(sw, _EPS * _EPS))
        wn_ref[...] = (w * inv_nw).astype(jnp.bfloat16)

    x = x_ref[...]
    sx = jnp.sum(x * x, axis=1, keepdims=True)
    inv_nx = lax.rsqrt(jnp.maximum(sx, _EPS * _EPS)) * s
    xn = (x * inv_nx).astype(jnp.bfloat16)

    raw = lax.dot_general(
        xn, wn_ref[...],
        dimension_numbers=(((1,), (1,)), ((), ())),
        preferred_element_type=jnp.float32)

    col0 = pl.program_id(0) * tile_c
    class_ids = lax.broadcasted_iota(jnp.int32, raw.shape, 1) + col0
    labels = lab_ref[...].reshape(-1, 1)
    o_ref[...] = jnp.where(class_ids == labels, raw - (s * m), raw)


def kernel(x, W, label, s=30.0, m=0.35, tile_c=2560, tile_b=256):
    B, F = x.shape
    C, F2 = W.shape
    assert F == F2

    tc = tile_c if C >= tile_c else _round_up(C, 128)
    tb = tile_b if B >= tile_b else _round_up(B, 8)
    Bp = _round_up(B, tb)
    Cp = _round_up(C, tc)
    x_p = x if Bp == B else jnp.pad(x, ((0, Bp - B), (0, 0)))
    W_p = W if Cp == C else jnp.pad(W, ((0, Cp - C), (0, 0)))
    lab = label.astype(jnp.int32).reshape(1, B)
    lab_p = lab if Bp == B else jnp.pad(lab, ((0, 0), (0, Bp - B)),
                                        constant_values=-1)

    out = pl.pallas_call(
        functools.partial(_cosface_fused_kernel, s=s, m=m, tile_c=tc),
        out_shape=jax.ShapeDtypeStruct((Bp, Cp), jnp.float32),
        grid=(Cp // tc, Bp // tb),
        in_specs=[
            pl.BlockSpec((1, tb), lambda j, i: (0, i)),
            pl.BlockSpec((tb, F), lambda j, i: (i, 0)),
            pl.BlockSpec((tc, F), lambda j, i: (j, 0)),
        ],
        out_specs=pl.BlockSpec((tb, tc), lambda j, i: (i, j)),
        scratch_shapes=[pltpu.VMEM((tc, F), jnp.bfloat16)],
        compiler_params=pltpu.CompilerParams(
            dimension_semantics=("parallel", "arbitrary"),
            vmem_limit_bytes=58 * 1024 * 1024,
        ),
    )(lab_p, x_p, W_p)
    return out[:B, :C]


# speedup vs baseline: 1.1782x; 1.1782x over previous
import functools

import jax
import jax.numpy as jnp
from jax import lax
from jax.experimental import pallas as pl
from jax.experimental.pallas import tpu as pltpu

_EPS = 1e-12


def _round_up(v, n):
    return (v + n - 1) // n * n


def _core_kernel(lab_ref, x_ref, w_hbm, o_hbm, xn_s, lab_s, cnt_s,
                 *, s, m, n_chunks, chunk, F, Bp):
    j = pl.program_id(0)

    x = x_ref[...]
    sx = jnp.sum(x * x, axis=1, keepdims=True)
    inv_nx = lax.rsqrt(jnp.maximum(sx, _EPS * _EPS)) * s
    xn_s[...] = (x * inv_nx).astype(jnp.bfloat16)
    lab_s[...] = lab_ref[...].reshape(Bp, 1)
    cnt_s[0] = 0

    def inner(w_ref, o_ref):
        c = cnt_s[0]
        cnt_s[0] = c + 1
        w = w_ref[...]
        sw = jnp.sum(w * w, axis=1, keepdims=True)
        inv_nw = lax.rsqrt(jnp.maximum(sw, _EPS * _EPS))
        wn = (w * inv_nw).astype(jnp.bfloat16)
        raw = lax.dot_general(
            xn_s[...], wn,
            dimension_numbers=(((1,), (1,)), ((), ())),
            preferred_element_type=jnp.float32)
        col0 = (j * n_chunks + c) * chunk
        class_ids = lax.broadcasted_iota(jnp.int32, raw.shape, 1) + col0
        o_ref[...] = jnp.where(class_ids == lab_s[...], raw - (s * m), raw)

    pltpu.emit_pipeline(
        inner,
        grid=(n_chunks,),
        in_specs=[pl.BlockSpec((chunk, F), lambda c: (j * n_chunks + c, 0))],
        out_specs=[pl.BlockSpec((Bp, chunk), lambda c: (0, j * n_chunks + c))],
    )(w_hbm, o_hbm)


def kernel(x, W, label, s=30.0, m=0.35, chunk=1280):
    B, F = x.shape
    C, F2 = W.shape
    assert F == F2

    ch = chunk if C >= 2 * chunk else _round_up(max(C // 2, 1), 128)
    Bp = _round_up(B, 8)
    Cp = _round_up(C, 2 * ch)
    n_chunks = Cp // (2 * ch)
    x_p = x if Bp == B else jnp.pad(x, ((0, Bp - B), (0, 0)))
    W_p = W if Cp == C else jnp.pad(W, ((0, Cp - C), (0, 0)))
    lab = label.astype(jnp.int32).reshape(1, B)
    lab_p = lab if Bp == B else jnp.pad(lab, ((0, 0), (0, Bp - B)),
                                        constant_values=-1)

    out = pl.pallas_call(
        functools.partial(_core_kernel, s=s, m=m, n_chunks=n_chunks,
                          chunk=ch, F=F, Bp=Bp),
        out_shape=jax.ShapeDtypeStruct((Bp, Cp), jnp.float32),
        grid=(2,),
        in_specs=[
            pl.BlockSpec((1, Bp), lambda j: (0, 0)),
            pl.BlockSpec((Bp, F), lambda j: (0, 0)),
            pl.BlockSpec(memory_space=pl.ANY),
        ],
        out_specs=pl.BlockSpec(memory_space=pl.ANY),
        scratch_shapes=[
            pltpu.VMEM((Bp, F), jnp.bfloat16),
            pltpu.VMEM((Bp, 1), jnp.int32),
            pltpu.SMEM((1,), jnp.int32),
        ],
        compiler_params=pltpu.CompilerParams(
            dimension_semantics=("parallel",),
            vmem_limit_bytes=58 * 1024 * 1024,
        ),
    )(lab_p, x_p, W_p)
    return out[:B, :C]


# speedup vs baseline: 1.3026x; 1.1056x over previous
import functools

import jax
import jax.numpy as jnp
from jax import lax
from jax.experimental import pallas as pl
from jax.experimental.pallas import tpu as pltpu

_EPS = 1e-12


def _round_up(v, n):
    return (v + n - 1) // n * n


def _core_kernel(lab_ref, x_ref, w_hbm, o_hbm, xn_s, lab_s, cnt_s,
                 *, s, m, n_chunks, chunk, F, Bp):
    j = pl.program_id(0)

    x = x_ref[...]
    sx = jnp.sum(x * x, axis=1, keepdims=True)
    inv_nx = lax.rsqrt(jnp.maximum(sx, _EPS * _EPS)) * s
    xn_s[...] = (x * inv_nx).astype(jnp.bfloat16)
    lab_s[...] = lab_ref[...].reshape(Bp, 1)
    cnt_s[0] = 0

    def inner(w_ref, o_ref):
        c = cnt_s[0]
        cnt_s[0] = c + 1
        w = w_ref[...]
        sw = jnp.sum(w * w, axis=1, keepdims=True)
        inv_nw = lax.rsqrt(jnp.maximum(sw, _EPS * _EPS))
        wn = (w * inv_nw).astype(jnp.bfloat16)
        raw = lax.dot_general(
            xn_s[...], wn,
            dimension_numbers=(((1,), (1,)), ((), ())),
            preferred_element_type=jnp.float32)
        col0 = (j * n_chunks + c) * chunk
        class_ids = lax.broadcasted_iota(jnp.int32, raw.shape, 1) + col0
        o_ref[...] = jnp.where(class_ids == lab_s[...], raw - (s * m), raw)

    pltpu.emit_pipeline(
        inner,
        grid=(n_chunks,),
        in_specs=[pl.BlockSpec((chunk, F), lambda c: (j * n_chunks + c, 0))],
        out_specs=[pl.BlockSpec((Bp, chunk), lambda c: (0, j * n_chunks + c))],
    )(w_hbm, o_hbm)


def kernel(x, W, label, s=30.0, m=0.35, chunk=2560):
    B, F = x.shape
    C, F2 = W.shape
    assert F == F2

    ch = chunk if C >= 2 * chunk else _round_up(max(C // 2, 1), 128)
    Bp = _round_up(B, 8)
    Cp = _round_up(C, 2 * ch)
    n_chunks = Cp // (2 * ch)
    x_p = x if Bp == B else jnp.pad(x, ((0, Bp - B), (0, 0)))
    W_p = W if Cp == C else jnp.pad(W, ((0, Cp - C), (0, 0)))
    lab = label.astype(jnp.int32).reshape(1, B)
    lab_p = lab if Bp == B else jnp.pad(lab, ((0, 0), (0, Bp - B)),
                                        constant_values=-1)

    out = pl.pallas_call(
        functools.partial(_core_kernel, s=s, m=m, n_chunks=n_chunks,
                          chunk=ch, F=F, Bp=Bp),
        out_shape=jax.ShapeDtypeStruct((Bp, Cp), jnp.float32),
        grid=(2,),
        in_specs=[
            pl.BlockSpec((1, Bp), lambda j: (0, 0)),
            pl.BlockSpec((Bp, F), lambda j: (0, 0)),
            pl.BlockSpec(memory_space=pl.ANY),
        ],
        out_specs=pl.BlockSpec(memory_space=pl.ANY),
        scratch_shapes=[
            pltpu.VMEM((Bp, F), jnp.bfloat16),
            pltpu.VMEM((Bp, 1), jnp.int32),
            pltpu.SMEM((1,), jnp.int32),
        ],
        compiler_params=pltpu.CompilerParams(
            dimension_semantics=("parallel",),
            vmem_limit_bytes=58 * 1024 * 1024,
        ),
    )(lab_p, x_p, W_p)
    return out[:B, :C]


# speedup vs baseline: 1.5356x; 1.1789x over previous
import functools

import jax
import jax.numpy as jnp
from jax import lax
from jax.experimental import pallas as pl
from jax.experimental.pallas import tpu as pltpu

_EPS = 1e-12


def _round_up(v, n):
    return (v + n - 1) // n * n


def _cosface_fused_kernel(lab_ref, x_ref, w_ref, o_ref, *, s, m, tile_c):
    x = x_ref[...]
    sx = jnp.sum(x * x, axis=1, keepdims=True)
    inv_nx = lax.rsqrt(jnp.maximum(sx, _EPS * _EPS)) * s
    xn = (x * inv_nx).astype(jnp.bfloat16)

    w = w_ref[...]
    sw = jnp.sum(w * w, axis=1, keepdims=True)
    inv_nw = lax.rsqrt(jnp.maximum(sw, _EPS * _EPS))
    wn = (w * inv_nw).astype(jnp.bfloat16)

    raw = lax.dot_general(
        xn, wn,
        dimension_numbers=(((1,), (1,)), ((), ())),
        preferred_element_type=jnp.float32)

    col0 = pl.program_id(0) * tile_c
    class_ids = lax.broadcasted_iota(jnp.int32, raw.shape, 1) + col0
    labels = lab_ref[...].reshape(-1, 1)
    o_ref[...] = jnp.where(class_ids == labels, raw - (s * m), raw)


def kernel(x, W, label, s=30.0, m=0.35, tile_c=5120):
    B, F = x.shape
    C, F2 = W.shape
    assert F == F2

    tc = tile_c if C >= tile_c else _round_up(C, 128)
    Bp = _round_up(B, 8)
    Cp = _round_up(C, tc)
    x_p = x if Bp == B else jnp.pad(x, ((0, Bp - B), (0, 0)))
    W_p = W if Cp == C else jnp.pad(W, ((0, Cp - C), (0, 0)))
    lab = label.astype(jnp.int32).reshape(1, B)
    lab_p = lab if Bp == B else jnp.pad(lab, ((0, 0), (0, Bp - B)),
                                        constant_values=-1)

    out = pl.pallas_call(
        functools.partial(_cosface_fused_kernel, s=s, m=m, tile_c=tc),
        out_shape=jax.ShapeDtypeStruct((Bp, Cp), jnp.float32),
        grid=(Cp // tc,),
        in_specs=[
            pl.BlockSpec((1, Bp), lambda j: (0, 0)),
            pl.BlockSpec((Bp, F), lambda j: (0, 0)),
            pl.BlockSpec((tc, F), lambda j: (j, 0)),
        ],
        out_specs=pl.BlockSpec((Bp, tc), lambda j: (0, j)),
        compiler_params=pltpu.CompilerParams(
            dimension_semantics=("parallel",),
            vmem_limit_bytes=58 * 1024 * 1024,
        ),
    )(lab_p, x_p, W_p)
    return out[:B, :C]
